# baseline (speedup 1.0000x reference)
import jax
import jax.numpy as jnp
from jax import lax
from jax.experimental import pallas as pl
from jax.experimental.pallas import tpu as pltpu
from jax.experimental.pallas import tpu_sc as plsc

_F = 26
_V = 100000
_D = 16
_B = 16384

_NC = 2
_NS = 16
_NW = _NC * _NS
_ROWS = _B * _F
_CW = 128
_NCHUNK = _ROWS // (_NW * _CW)


def _body(x_hbm, tab_hbm, out_hbm, idx_v, rows_v, sem):
    wid = lax.axis_index("s") * _NC + lax.axis_index("c")
    row0 = wid * _NCHUNK
    pltpu.sync_copy(x_hbm.at[pl.ds(row0, _NCHUNK)], idx_v)
    lane = lax.iota(jnp.int32, 16)

    def adjust(j, carry):
        base = (row0 + j) * _CW
        for l in range(_CW // 16):
            pos = base + l * 16 + lane
            f = lax.rem(pos, _F)
            idx_v[j, pl.ds(l * 16, 16)] = idx_v[j, pl.ds(l * 16, 16)] + f * _V
        return carry

    lax.fori_loop(0, _NCHUNK, adjust, 0)

    def gather(j, carry):
        pltpu.async_copy(tab_hbm.at[idx_v.at[j]], rows_v, sem).wait()
        pltpu.sync_copy(rows_v, out_hbm.at[pl.ds((row0 + j) * _CW, _CW)])
        return carry

    lax.fori_loop(0, _NCHUNK, gather, 0)


@jax.jit
def _mce(xf, tab):
    mesh = plsc.VectorSubcoreMesh(core_axis_name="c", subcore_axis_name="s",
                                  num_cores=_NC, num_subcores=_NS)
    return pl.kernel(
        _body,
        out_type=jax.ShapeDtypeStruct((_ROWS, _D), jnp.float32),
        mesh=mesh,
        scratch_types=[
            pltpu.VMEM((_NCHUNK, _CW), jnp.int32),
            pltpu.VMEM((_CW, _D), jnp.float32),
            pltpu.SemaphoreType.DMA,
        ],
        compiler_params=pltpu.CompilerParams(use_tc_tiling_on_sc=False),
    )(xf, tab)


def kernel(X, tables):
    xf = X.reshape(_ROWS // _CW, _CW).astype(jnp.int32)
    tab = tables.reshape(_F * _V, _D)
    return _mce(xf, tab).reshape(_B, _F * _D)


# speedup vs baseline: 1.0605x; 1.0605x over previous
import jax
import jax.numpy as jnp
from jax import lax
from jax.experimental import pallas as pl
from jax.experimental.pallas import tpu as pltpu
from jax.experimental.pallas import tpu_sc as plsc

_F = 26
_V = 100000
_D = 16
_B = 16384

_NC = 2
_NS = 16
_NW = _NC * _NS
_ROWS = _B * _F
_CW = 128
_NCHUNK = _ROWS // (_NW * _CW)
_K = 13
_NG = _NCHUNK // _K


def _body(x_hbm, tab_hbm, out_hbm, idx_v, rows_a, rows_b, sem_a, sem_b,
          osem_a, osem_b):
    wid = lax.axis_index("s") * _NC + lax.axis_index("c")
    row0 = wid * _NCHUNK
    pltpu.sync_copy(x_hbm.at[pl.ds(row0, _NCHUNK)], idx_v)
    lane = lax.iota(jnp.int32, 16)

    def adjust_row(j, carry):
        base = (row0 + j) * _CW
        for l in range(_CW // 16):
            pos = base + l * 16 + lane
            f = lax.rem(pos, _F)
            idx_v[j, pl.ds(l * 16, 16)] = idx_v[j, pl.ds(l * 16, 16)] + f * _V
        return carry

    def adjust_group(g):
        lax.fori_loop(g * _K, (g + 1) * _K, adjust_row, 0)

    def out_slice(g):
        return out_hbm.at[pl.ds((row0 + g * _K) * _CW, _K * _CW)]

    bufs = (rows_a, rows_b)
    gsems = (sem_a, sem_b)
    osems = (osem_a, osem_b)

    adjust_group(0)
    for g in range(_NG):
        buf, gsem, osem = bufs[g % 2], gsems[g % 2], osems[g % 2]
        if g >= 2:
            pltpu.make_async_copy(buf, out_slice(g - 2), osem).wait()
        handles = [
            pltpu.async_copy(
                tab_hbm.at[idx_v.at[g * _K + jj]],
                buf.at[pl.ds(jj * _CW, _CW)], gsem)
            for jj in range(_K)
        ]
        if g + 1 < _NG:
            adjust_group(g + 1)
        for h in handles:
            h.wait()
        pltpu.async_copy(buf, out_slice(g), osem)
    for g in (_NG - 2, _NG - 1):
        pltpu.make_async_copy(bufs[g % 2], out_slice(g), osems[g % 2]).wait()


@jax.jit
def _mce(xf, tab):
    mesh = plsc.VectorSubcoreMesh(core_axis_name="c", subcore_axis_name="s",
                                  num_cores=_NC, num_subcores=_NS)
    return pl.kernel(
        _body,
        out_type=jax.ShapeDtypeStruct((_ROWS, _D), jnp.float32),
        mesh=mesh,
        scratch_types=[
            pltpu.VMEM((_NCHUNK, _CW), jnp.int32),
            pltpu.VMEM((_K * _CW, _D), jnp.float32),
            pltpu.VMEM((_K * _CW, _D), jnp.float32),
            pltpu.SemaphoreType.DMA,
            pltpu.SemaphoreType.DMA,
            pltpu.SemaphoreType.DMA,
            pltpu.SemaphoreType.DMA,
        ],
        compiler_params=pltpu.CompilerParams(use_tc_tiling_on_sc=False),
    )(xf, tab)


def kernel(X, tables):
    xf = X.reshape(_ROWS // _CW, _CW).astype(jnp.int32)
    tab = tables.reshape(_F * _V, _D)
    return _mce(xf, tab).reshape(_B, _F * _D)


# speedup vs baseline: 1.0608x; 1.0003x over previous
import jax
import jax.numpy as jnp
from jax import lax
from jax.experimental import pallas as pl
from jax.experimental.pallas import tpu as pltpu
from jax.experimental.pallas import tpu_sc as plsc

_F = 26
_V = 100000
_D = 16
_B = 16384

_NC = 2
_NS = 16
_NW = _NC * _NS
_ROWS = _B * _F
_BW = _B // _NW
_CW = 128
_NCHUNK = _BW * _F // _CW
_K = 13
_NG = _NCHUNK // _K


def _body(x_hbm, tab_hbm, out_hbm, xraw_v, idx_v, rows_a, rows_b,
          sem_a, sem_b, osem_a, osem_b):
    wid = lax.axis_index("s") * _NC + lax.axis_index("c")
    row0 = wid * _NCHUNK
    pltpu.sync_copy(x_hbm.at[pl.ds(wid * _BW, _BW)], xraw_v)
    lane = lax.iota(jnp.int32, 16)

    def repack_row(j, carry):
        for l in range(_CW // 16):
            p = j * _CW + l * 16 + lane
            r = lax.div(p, _F)
            c = p - r * _F
            v = plsc.load_gather(xraw_v, [r, c])
            idx_v[j, pl.ds(l * 16, 16)] = v + c * _V
        return carry

    def repack_group(g):
        lax.fori_loop(g * _K, (g + 1) * _K, repack_row, 0)

    def out_slice(g):
        return out_hbm.at[pl.ds((row0 + g * _K) * _CW, _K * _CW)]

    bufs = (rows_a, rows_b)
    gsems = (sem_a, sem_b)
    osems = (osem_a, osem_b)

    repack_group(0)
    for g in range(_NG):
        buf, gsem, osem = bufs[g % 2], gsems[g % 2], osems[g % 2]
        if g >= 2:
            pltpu.make_async_copy(buf, out_slice(g - 2), osem).wait()
        handles = [
            pltpu.async_copy(
                tab_hbm.at[idx_v.at[g * _K + jj]],
                buf.at[pl.ds(jj * _CW, _CW)], gsem)
            for jj in range(_K)
        ]
        if g + 1 < _NG:
            repack_group(g + 1)
        for h in handles:
            h.wait()
        pltpu.async_copy(buf, out_slice(g), osem)
    for g in (_NG - 2, _NG - 1):
        pltpu.make_async_copy(bufs[g % 2], out_slice(g), osems[g % 2]).wait()


@jax.jit
def _mce(x, tab):
    mesh = plsc.VectorSubcoreMesh(core_axis_name="c", subcore_axis_name="s",
                                  num_cores=_NC, num_subcores=_NS)
    return pl.kernel(
        _body,
        out_type=jax.ShapeDtypeStruct((_ROWS, _D), jnp.float32),
        mesh=mesh,
        scratch_types=[
            pltpu.VMEM((_BW, _F), jnp.int32),
            pltpu.VMEM((_NCHUNK, _CW), jnp.int32),
            pltpu.VMEM((_K * _CW, _D), jnp.float32),
            pltpu.VMEM((_K * _CW, _D), jnp.float32),
            pltpu.SemaphoreType.DMA,
            pltpu.SemaphoreType.DMA,
            pltpu.SemaphoreType.DMA,
            pltpu.SemaphoreType.DMA,
        ],
        compiler_params=pltpu.CompilerParams(use_tc_tiling_on_sc=False,
                                             needs_layout_passes=False),
    )(x, tab)


def kernel(X, tables):
    tab = tables.reshape(_F * _V, _D)
    return _mce(X.astype(jnp.int32), tab).reshape(_B, _F * _D)


# speedup vs baseline: 4.6022x; 4.3386x over previous
import jax
import jax.numpy as jnp
from jax import lax
from jax.experimental import pallas as pl
from jax.experimental.pallas import tpu as pltpu
from jax.experimental.pallas import tpu_sc as plsc

_F = 26
_V = 100000
_D = 16
_B = 16384

_NC = 2
_NS = 16
_NW = _NC * _NS
_NR = _F * _D
_RPW = _NR // _NW
_BH = _B // 2


def _body(x_hbm, tab_hbm, out_hbm, trow_v, xcol_v, ob_a, ob_b, osem_a, osem_b):
    wid = lax.axis_index("s") * _NC + lax.axis_index("c")

    obufs = (ob_a, ob_b)
    osems = (osem_a, osem_b)

    def out_slice(r, h):
        return out_hbm.at[pl.ds(r * 128 + h * 64, 64), :]

    for k in range(_RPW):
        r = wid * _RPW + k
        f = lax.div(r, _D)
        d = lax.rem(r, _D)
        pltpu.sync_copy(tab_hbm.at[f, d], trow_v)
        for h in range(2):
            slot = (2 * k + h) % 2
            obuf, osem = obufs[slot], osems[slot]
            if 2 * k + h >= 2:
                pltpu.make_async_copy(obuf, out_slice(r, h), osem).wait()
            pltpu.sync_copy(x_hbm.at[f, pl.ds(h * _BH, _BH)], xcol_v)

            def extract(j, carry):
                for l in range(8):
                    idx16 = xcol_v[pl.ds(j * 128 + l * 16, 16)]
                    obuf[j, pl.ds(l * 16, 16)] = plsc.load_gather(
                        trow_v, [idx16])
                return carry

            lax.fori_loop(0, 64, extract, 0, unroll=2)
            pltpu.async_copy(obuf, out_slice(r, h), osem)
    for h in range(2):
        r = wid * _RPW + _RPW - 1
        pltpu.make_async_copy(obufs[h], out_slice(r, h), osems[h]).wait()


@jax.jit
def _mce(xT, tab_t):
    mesh = plsc.VectorSubcoreMesh(core_axis_name="c", subcore_axis_name="s",
                                  num_cores=_NC, num_subcores=_NS)
    return pl.kernel(
        _body,
        out_type=jax.ShapeDtypeStruct((_NR * _B // 128, 128), jnp.float32),
        mesh=mesh,
        scratch_types=[
            pltpu.VMEM((_V,), jnp.float32),
            pltpu.VMEM((_BH,), jnp.int32),
            pltpu.VMEM((64, 128), jnp.float32),
            pltpu.VMEM((64, 128), jnp.float32),
            pltpu.SemaphoreType.DMA,
            pltpu.SemaphoreType.DMA,
        ],
        compiler_params=pltpu.CompilerParams(use_tc_tiling_on_sc=True,
                                             needs_layout_passes=False),
    )(xT, tab_t)


def kernel(X, tables):
    tab_t = jnp.transpose(tables, (0, 2, 1))
    xT = X.astype(jnp.int32).T
    res = _mce(xT, tab_t)
    return res.reshape(_NR, _B).T
